# baseline (speedup 1.0000x reference)
import functools

import jax
import jax.numpy as jnp
from jax import lax
from jax.experimental import pallas as pl
from jax.experimental.pallas import tpu as pltpu
from jax.experimental.pallas import tpu_sc as plsc

NACC = 10240
LCH = 128
NT = 32
NS = 16
SEG = NACC // NS

_SC_PARAMS = pltpu.CompilerParams(needs_layout_passes=False)


def _make_deg_kernel(tc):
    mesh = plsc.VectorSubcoreMesh(core_axis_name="c", subcore_axis_name="s")

    @functools.partial(
        pl.kernel, mesh=mesh,
        out_type=jax.ShapeDtypeStruct((NT, 2, NACC), jnp.float32),
        compiler_params=_SC_PARAMS,
        scratch_types=[
            pltpu.VMEM((tc,), jnp.int32),
            pltpu.VMEM((tc,), jnp.int32),
            pltpu.VMEM((NACC,), jnp.float32),
            pltpu.VMEM((NACC,), jnp.float32),
            pltpu.SemaphoreType.DMA,
        ],
    )
    def deg_kernel(idx_hbm, zero_hbm, out_hbm, i0, i1, a0, a1, sem):
        cid = lax.axis_index("c")
        sid = lax.axis_index("s")
        wid = sid * 2 + cid
        accs = [a0, a1]
        idxs = [i0, i1]
        ones = jnp.ones((16,), jnp.float32)
        descs = [pltpu.async_copy(idx_hbm.at[0, wid], i0, sem),
                 pltpu.async_copy(idx_hbm.at[2, wid], i1, sem),
                 pltpu.async_copy(zero_hbm, a0, sem),
                 pltpu.async_copy(zero_hbm, a1, sem)]
        for d in descs:
            d.wait()

        def sb(j, c):
            for u in range(2):
                for a in range(2):
                    v = idxs[a][pl.ds(j * 32 + u * 16, 16)]
                    plsc.addupdate_scatter(accs[a], [v], ones)
            return c
        lax.fori_loop(0, tc // 32, sb, 0)
        for a in range(2):
            pltpu.sync_copy(accs[a], out_hbm.at[wid, a])

    return deg_kernel


def _make_edge_kernel(tc):
    mesh = plsc.VectorSubcoreMesh(core_axis_name="c", subcore_axis_name="s")

    @functools.partial(
        pl.kernel, mesh=mesh,
        out_type=(jax.ShapeDtypeStruct((NT, 2, NACC), jnp.float32),
                  jax.ShapeDtypeStruct((2, 2, NACC), jnp.float32)),
        compiler_params=_SC_PARAMS,
        scratch_types=[
            pltpu.VMEM((tc,), jnp.int32),
            pltpu.VMEM((tc,), jnp.int32),
            pltpu.VMEM((tc,), jnp.int32),
            pltpu.VMEM((tc,), jnp.int32),
            pltpu.VMEM((tc,), jnp.float32),
            pltpu.VMEM((tc,), jnp.float32),
            pltpu.VMEM((tc,), jnp.float32),
            pltpu.VMEM((NACC,), jnp.float32),
            pltpu.VMEM((NACC,), jnp.float32),
            pltpu.VMEM((SEG,), jnp.float32),
            pltpu.VMEM_SHARED((NACC,), jnp.float32),
            pltpu.VMEM_SHARED((NACC,), jnp.float32),
            pltpu.SemaphoreType.DMA,
            pltpu.SemaphoreType.DMA,
            pltpu.SemaphoreType.DMA,
            pltpu.SemaphoreType.DMA,
        ],
    )
    def edge_kernel(q0_hbm, q1_hbm, idx_hbm, ones_hbm, zero_hbm,
                    t_out, cnt_out, s0, s1, d0, d1, v0, v1, ones_v,
                    a0, a1, zseg_v, shc0, shc1, semA, semG0, semG1, semC):
        cid = lax.axis_index("c")
        sid = lax.axis_index("s")
        wid = sid * 2 + cid
        didx = [d0, d1]
        vals = [v0, v1]
        accs = [a0, a1]
        descs = [pltpu.async_copy(idx_hbm.at[0, wid], s0, semA),
                 pltpu.async_copy(idx_hbm.at[1, wid], d0, semA),
                 pltpu.async_copy(idx_hbm.at[2, wid], s1, semA),
                 pltpu.async_copy(idx_hbm.at[3, wid], d1, semA),
                 pltpu.async_copy(ones_hbm, ones_v, semA),
                 pltpu.async_copy(zero_hbm, a0, semA),
                 pltpu.async_copy(zero_hbm, a1, semA)]
        for i in range(SEG // 16):
            zseg_v[pl.ds(i * 16, 16)] = jnp.zeros((16,), jnp.float32)
        for d in descs:
            d.wait()
        pltpu.sync_copy(zseg_v, shc0.at[pl.ds(sid * SEG, SEG)])
        pltpu.sync_copy(zseg_v, shc1.at[pl.ds(sid * SEG, SEG)])
        plsc.subcore_barrier()
        g0 = pltpu.async_copy(q0_hbm.at[s0], v0, semG0)
        g1 = pltpu.async_copy(q1_hbm.at[s1], v1, semG1)
        c0 = pltpu.async_copy(ones_v, shc0.at[d0], semC, add=True)
        c1 = pltpu.async_copy(ones_v, shc1.at[d1], semC, add=True)
        gsems = [g0, g1]
        for s in range(2):
            gsems[s].wait()

            def eb(j, c, _s=s):
                for u in range(2):
                    sl = pl.ds(j * 32 + u * 16, 16)
                    plsc.addupdate_scatter(accs[_s], [didx[_s][sl]],
                                           vals[_s][sl])
                return c
            lax.fori_loop(0, tc // 32, eb, 0)
        for s in range(2):
            pltpu.sync_copy(accs[s], t_out.at[wid, s])
        c0.wait()
        c1.wait()
        plsc.subcore_barrier()

        @pl.when(sid == 0)
        def _():
            pltpu.sync_copy(shc0, cnt_out.at[cid, 0])
            pltpu.sync_copy(shc1, cnt_out.at[cid, 1])

    return edge_kernel


def _mv_kernel(dx_ref, px_ref, wdp_ref, w1_ref, wpd_ref, w2_ref, s_ref):
    u1 = jnp.dot(wdp_ref[...], w1_ref[...], preferred_element_type=jnp.float32)
    u2 = jnp.dot(wpd_ref[...], w2_ref[...], preferred_element_type=jnp.float32)
    s_ref[0, :] = jnp.dot(dx_ref[...], u1, preferred_element_type=jnp.float32)[:, 0]
    s_ref[1, :] = jnp.dot(px_ref[...], u2, preferred_element_type=jnp.float32)[:, 0]


def _q_kernel(s_ref, deg_ref, q_ref):
    deg = jnp.sum(deg_ref[...], axis=0)
    r = lax.rsqrt(jnp.maximum(deg, 1.0))
    q_ref[0, :] = s_ref[0] * r[0]
    q_ref[1, :] = s_ref[1] * r[1]


def _fin_kernel(t_ref, cnt_ref, bdp_ref, bpd_ref, w1_ref, w2_ref, bl_ref, o_ref):
    c1 = (jnp.sum(bdp_ref[...] * w1_ref[...])
          + jnp.sum(bpd_ref[...] * w2_ref[...]) + bl_ref[0, 0])
    t = jnp.sum(t_ref[...], axis=0)
    cnt = cnt_ref[0] + cnt_ref[1]
    r0 = lax.rsqrt(jnp.maximum(cnt[0], 1.0))
    r1 = lax.rsqrt(jnp.maximum(cnt[1], 1.0))
    z = t[0] * r0 + t[1] * r1 + c1
    o_ref[...] = 1.0 / (1.0 + jnp.exp(-z))


def kernel(drug_x, protein_x, edge_index, rev_edge_index, W_drug_lin,
           b_drug_lin, W_protein_lin, b_protein_lin, conv_W_dp, conv_b_dp,
           conv_W_pd, conv_b_pd, W_link, b_link):
    n = drug_x.shape[0]
    d_h = conv_W_dp.shape[2]
    e = edge_index.shape[1]
    tc = -(-e // (NT * LCH)) * LCH
    epad = NT * tc

    w1 = W_link[:d_h]
    w2 = W_link[d_h:]
    wdp = conv_W_dp[-1]
    wpd = conv_W_pd[-1]
    zeros_acc = jnp.zeros((NACC,), jnp.float32)

    def prep(v):
        pad = jnp.full((epad - e,), n, jnp.int32)
        return jnp.concatenate([v.astype(jnp.int32), pad]).reshape(NT, tc)

    idx_all = jnp.stack([prep(edge_index[0]), prep(edge_index[1]),
                         prep(rev_edge_index[0]), prep(rev_edge_index[1])])

    ones_e = jnp.ones((tc,), jnp.float32)
    deg_part = _make_deg_kernel(tc)(idx_all, zeros_acc)

    blk = 1024
    nb = NACC // blk
    s = pl.pallas_call(
        _mv_kernel,
        grid=(nb,),
        in_specs=[
            pl.BlockSpec((blk, drug_x.shape[1]), lambda i: (i, 0)),
            pl.BlockSpec((blk, protein_x.shape[1]), lambda i: (i, 0)),
            pl.BlockSpec(wdp.shape, lambda i: (0, 0)),
            pl.BlockSpec(w1.shape, lambda i: (0, 0)),
            pl.BlockSpec(wpd.shape, lambda i: (0, 0)),
            pl.BlockSpec(w2.shape, lambda i: (0, 0)),
        ],
        out_specs=pl.BlockSpec((2, blk), lambda i: (0, i)),
        out_shape=jax.ShapeDtypeStruct((2, NACC), jnp.float32),
    )(drug_x, protein_x, wdp, w1, wpd, w2)

    q = pl.pallas_call(
        _q_kernel,
        grid=(nb,),
        in_specs=[
            pl.BlockSpec((2, blk), lambda i: (0, i)),
            pl.BlockSpec((NT, 2, blk), lambda i: (0, 0, i)),
        ],
        out_specs=pl.BlockSpec((2, blk), lambda i: (0, i)),
        out_shape=jax.ShapeDtypeStruct((2, NACC), jnp.float32),
    )(s, deg_part)

    t_part, cnt_part = _make_edge_kernel(tc)(q[0], q[1], idx_all, ones_e,
                                             zeros_acc)

    out_full = pl.pallas_call(
        _fin_kernel,
        out_shape=jax.ShapeDtypeStruct((NACC,), jnp.float32),
    )(t_part, cnt_part,
      conv_b_dp[-1].reshape(2, d_h // 2), conv_b_pd[-1].reshape(2, d_h // 2),
      w1.reshape(2, d_h // 2), w2.reshape(2, d_h // 2),
      b_link.reshape(1, 1))

    return out_full[:n].reshape(n, 1)


# speedup vs baseline: 1.2634x; 1.2634x over previous
import functools

import jax
import jax.numpy as jnp
from jax import lax
from jax.experimental import pallas as pl
from jax.experimental.pallas import tpu as pltpu
from jax.experimental.pallas import tpu_sc as plsc

NACC = 10240
LCH = 128
NT = 32

_SC_PARAMS = pltpu.CompilerParams(needs_layout_passes=False)


def _make_deg_kernel(tc):
    mesh = plsc.VectorSubcoreMesh(core_axis_name="c", subcore_axis_name="s")

    @functools.partial(
        pl.kernel, mesh=mesh,
        out_type=jax.ShapeDtypeStruct((NT, 4, NACC), jnp.float32),
        compiler_params=_SC_PARAMS,
        scratch_types=[
            pltpu.VMEM((tc,), jnp.int32),
            pltpu.VMEM((tc,), jnp.int32),
            pltpu.VMEM((tc,), jnp.int32),
            pltpu.VMEM((tc,), jnp.int32),
            pltpu.VMEM((NACC,), jnp.float32),
            pltpu.VMEM((NACC,), jnp.float32),
            pltpu.VMEM((NACC,), jnp.float32),
            pltpu.VMEM((NACC,), jnp.float32),
            pltpu.SemaphoreType.DMA,
        ],
    )
    def deg_kernel(idx_hbm, zero_hbm, out_hbm, i0, i1, i2, i3,
                   a0, a1, a2, a3, sem):
        cid = lax.axis_index("c")
        sid = lax.axis_index("s")
        wid = sid * 2 + cid
        accs = [a0, a1, a2, a3]
        idxs = [i0, i1, i2, i3]
        ones = jnp.ones((16,), jnp.float32)
        zero = jnp.zeros((16,), jnp.float32)
        descs = [pltpu.async_copy(idx_hbm.at[a, wid], idxs[a], sem)
                 for a in range(4)]

        def zb(j, c):
            for a in range(4):
                accs[a][pl.ds(j * 16, 16)] = zero
            return c
        lax.fori_loop(0, NACC // 16, zb, 0)
        for d in descs:
            d.wait()

        def sb(j, c):
            for u in range(2):
                for a in range(4):
                    v = idxs[a][pl.ds(j * 32 + u * 16, 16)]
                    plsc.addupdate_scatter(accs[a], [v], ones)
            return c
        lax.fori_loop(0, tc // 32, sb, 0)
        for a in range(4):
            pltpu.sync_copy(accs[a], out_hbm.at[wid, a])

    return deg_kernel


def _make_edge_kernel(tc):
    mesh = plsc.VectorSubcoreMesh(core_axis_name="c", subcore_axis_name="s")

    @functools.partial(
        pl.kernel, mesh=mesh,
        out_type=jax.ShapeDtypeStruct((NT, 2, NACC), jnp.float32),
        compiler_params=_SC_PARAMS,
        scratch_types=[
            pltpu.VMEM((tc,), jnp.int32),
            pltpu.VMEM((tc,), jnp.int32),
            pltpu.VMEM((tc,), jnp.int32),
            pltpu.VMEM((tc,), jnp.int32),
            pltpu.VMEM((NACC,), jnp.float32),
            pltpu.VMEM((NACC,), jnp.float32),
            pltpu.VMEM((NACC,), jnp.float32),
            pltpu.VMEM((NACC,), jnp.float32),
            pltpu.SemaphoreType.DMA,
        ],
    )
    def edge_kernel(q_hbm, idx_hbm, zero_hbm, out_hbm, s0, s1, d0, d1,
                    q0, q1, a0, a1, sem):
        cid = lax.axis_index("c")
        sid = lax.axis_index("s")
        wid = sid * 2 + cid
        sidx = [s0, s1]
        didx = [d0, d1]
        qv = [q0, q1]
        accs = [a0, a1]
        descs = []
        for s in range(2):
            descs.append(pltpu.async_copy(idx_hbm.at[2 * s, wid], sidx[s], sem))
            descs.append(pltpu.async_copy(idx_hbm.at[2 * s + 1, wid], didx[s], sem))
            descs.append(pltpu.async_copy(q_hbm.at[s], qv[s], sem))
            descs.append(pltpu.async_copy(zero_hbm, accs[s], sem))
        for d in descs:
            d.wait()

        def eb(j, c):
            for u in range(2):
                for s in range(2):
                    sv = sidx[s][pl.ds(j * 32 + u * 16, 16)]
                    vals = plsc.load_gather(qv[s], [sv])
                    dv = didx[s][pl.ds(j * 32 + u * 16, 16)]
                    plsc.addupdate_scatter(accs[s], [dv], vals)
            return c
        lax.fori_loop(0, tc // 32, eb, 0)
        for s in range(2):
            pltpu.sync_copy(accs[s], out_hbm.at[wid, s])

    return edge_kernel


def _mv_kernel(dx_ref, px_ref, wdp_ref, w1_ref, wpd_ref, w2_ref, s_ref):
    u1 = jnp.dot(wdp_ref[...], w1_ref[...], preferred_element_type=jnp.float32)
    u2 = jnp.dot(wpd_ref[...], w2_ref[...], preferred_element_type=jnp.float32)
    s_ref[0, :] = jnp.dot(dx_ref[...], u1, preferred_element_type=jnp.float32)[:, 0]
    s_ref[1, :] = jnp.dot(px_ref[...], u2, preferred_element_type=jnp.float32)[:, 0]


def _q_kernel(s_ref, deg_ref, q_ref):
    deg = jnp.sum(deg_ref[...], axis=0)
    r = lax.rsqrt(jnp.maximum(deg, 1.0))
    q_ref[0, :] = s_ref[0] * r[0]
    q_ref[1, :] = s_ref[1] * r[2]
    q_ref[2, :] = r[1]
    q_ref[3, :] = r[3]


def _fin_kernel(t_ref, q_ref, bdp_ref, bpd_ref, w1_ref, w2_ref, bl_ref, o_ref):
    c1 = (jnp.sum(bdp_ref[...] * w1_ref[...])
          + jnp.sum(bpd_ref[...] * w2_ref[...]) + bl_ref[0, 0])
    t = jnp.sum(t_ref[...], axis=0)
    z = t[0] * q_ref[2, :] + t[1] * q_ref[3, :] + c1
    o_ref[...] = 1.0 / (1.0 + jnp.exp(-z))


def kernel(drug_x, protein_x, edge_index, rev_edge_index, W_drug_lin,
           b_drug_lin, W_protein_lin, b_protein_lin, conv_W_dp, conv_b_dp,
           conv_W_pd, conv_b_pd, W_link, b_link):
    n = drug_x.shape[0]
    d_h = conv_W_dp.shape[2]
    e = edge_index.shape[1]
    tc = -(-e // (NT * LCH)) * LCH
    epad = NT * tc

    w1 = W_link[:d_h]
    w2 = W_link[d_h:]
    wdp = conv_W_dp[-1]
    wpd = conv_W_pd[-1]
    zeros_acc = jnp.zeros((NACC,), jnp.float32)

    def prep(v):
        pad = jnp.full((epad - e,), n, jnp.int32)
        return jnp.concatenate([v.astype(jnp.int32), pad]).reshape(NT, tc)

    idx_all = jnp.stack([prep(edge_index[0]), prep(edge_index[1]),
                         prep(rev_edge_index[0]), prep(rev_edge_index[1])])

    deg_part = _make_deg_kernel(tc)(idx_all, zeros_acc)

    blk = 1024
    nb = NACC // blk
    s = pl.pallas_call(
        _mv_kernel,
        grid=(nb,),
        in_specs=[
            pl.BlockSpec((blk, drug_x.shape[1]), lambda i: (i, 0)),
            pl.BlockSpec((blk, protein_x.shape[1]), lambda i: (i, 0)),
            pl.BlockSpec(wdp.shape, lambda i: (0, 0)),
            pl.BlockSpec(w1.shape, lambda i: (0, 0)),
            pl.BlockSpec(wpd.shape, lambda i: (0, 0)),
            pl.BlockSpec(w2.shape, lambda i: (0, 0)),
        ],
        out_specs=pl.BlockSpec((2, blk), lambda i: (0, i)),
        out_shape=jax.ShapeDtypeStruct((2, NACC), jnp.float32),
    )(drug_x, protein_x, wdp, w1, wpd, w2)

    q = pl.pallas_call(
        _q_kernel,
        grid=(nb,),
        in_specs=[
            pl.BlockSpec((2, blk), lambda i: (0, i)),
            pl.BlockSpec((NT, 4, blk), lambda i: (0, 0, i)),
        ],
        out_specs=pl.BlockSpec((4, blk), lambda i: (0, i)),
        out_shape=jax.ShapeDtypeStruct((4, NACC), jnp.float32),
    )(s, deg_part)

    t_part = _make_edge_kernel(tc)(q[:2], idx_all, zeros_acc)

    out_full = pl.pallas_call(
        _fin_kernel,
        out_shape=jax.ShapeDtypeStruct((NACC,), jnp.float32),
    )(t_part, q,
      conv_b_dp[-1].reshape(2, d_h // 2), conv_b_pd[-1].reshape(2, d_h // 2),
      w1.reshape(2, d_h // 2), w2.reshape(2, d_h // 2),
      b_link.reshape(1, 1))

    return out_full[:n].reshape(n, 1)


# speedup vs baseline: 1.2925x; 1.0230x over previous
import functools

import jax
import jax.numpy as jnp
from jax import lax
from jax.experimental import pallas as pl
from jax.experimental.pallas import tpu as pltpu
from jax.experimental.pallas import tpu_sc as plsc

NACC = 10240
LCH = 128
NT = 32

_SC_PARAMS = pltpu.CompilerParams(needs_layout_passes=False)


def _make_deg_kernel(tc):
    mesh = plsc.VectorSubcoreMesh(core_axis_name="c", subcore_axis_name="s")

    @functools.partial(
        pl.kernel, mesh=mesh,
        out_type=jax.ShapeDtypeStruct((NT, 4, NACC), jnp.float32),
        compiler_params=_SC_PARAMS,
        scratch_types=[
            pltpu.VMEM((tc,), jnp.int32),
            pltpu.VMEM((tc,), jnp.int32),
            pltpu.VMEM((tc,), jnp.int32),
            pltpu.VMEM((tc,), jnp.int32),
            pltpu.VMEM((NACC,), jnp.float32),
            pltpu.VMEM((NACC,), jnp.float32),
            pltpu.VMEM((NACC,), jnp.float32),
            pltpu.VMEM((NACC,), jnp.float32),
            pltpu.SemaphoreType.DMA,
        ],
    )
    def deg_kernel(idx_hbm, out_hbm, i0, i1, i2, i3,
                   a0, a1, a2, a3, sem):
        cid = lax.axis_index("c")
        sid = lax.axis_index("s")
        wid = sid * 2 + cid
        accs = [a0, a1, a2, a3]
        idxs = [i0, i1, i2, i3]
        ones = jnp.ones((16,), jnp.float32)
        zero = jnp.zeros((16,), jnp.float32)
        descs = [pltpu.async_copy(idx_hbm.at[a, wid], idxs[a], sem)
                 for a in range(4)]

        def zb(j, c):
            for a in range(4):
                accs[a][pl.ds(j * 16, 16)] = zero
            return c
        lax.fori_loop(0, NACC // 16, zb, 0)
        for d in descs:
            d.wait()

        def sb(j, c):
            for u in range(2):
                for a in range(4):
                    v = idxs[a][pl.ds(j * 32 + u * 16, 16)]
                    plsc.addupdate_scatter(accs[a], [v], ones)
            return c
        lax.fori_loop(0, tc // 32, sb, 0)
        for a in range(4):
            pltpu.sync_copy(accs[a], out_hbm.at[wid, a])

    return deg_kernel


def _make_edge_kernel(tc):
    mesh = plsc.VectorSubcoreMesh(core_axis_name="c", subcore_axis_name="s")

    @functools.partial(
        pl.kernel, mesh=mesh,
        out_type=jax.ShapeDtypeStruct((NT, 2, NACC), jnp.float32),
        compiler_params=_SC_PARAMS,
        scratch_types=[
            pltpu.VMEM((tc,), jnp.int32),
            pltpu.VMEM((tc,), jnp.int32),
            pltpu.VMEM((tc,), jnp.int32),
            pltpu.VMEM((tc,), jnp.int32),
            pltpu.VMEM((NACC,), jnp.float32),
            pltpu.VMEM((NACC,), jnp.float32),
            pltpu.VMEM((NACC,), jnp.float32),
            pltpu.VMEM((NACC,), jnp.float32),
            pltpu.SemaphoreType.DMA,
        ],
    )
    def edge_kernel(q_hbm, idx_hbm, out_hbm, s0, s1, d0, d1,
                    q0, q1, a0, a1, sem):
        cid = lax.axis_index("c")
        sid = lax.axis_index("s")
        wid = sid * 2 + cid
        sidx = [s0, s1]
        didx = [d0, d1]
        qv = [q0, q1]
        accs = [a0, a1]
        zero = jnp.zeros((16,), jnp.float32)
        descs = []
        for s in range(2):
            descs.append(pltpu.async_copy(idx_hbm.at[2 * s, wid], sidx[s], sem))
            descs.append(pltpu.async_copy(idx_hbm.at[2 * s + 1, wid], didx[s], sem))
            descs.append(pltpu.async_copy(q_hbm.at[s], qv[s], sem))

        def zb(j, c):
            for s in range(2):
                accs[s][pl.ds(j * 16, 16)] = zero
            return c
        lax.fori_loop(0, NACC // 16, zb, 0)
        for d in descs:
            d.wait()

        def eb(j, c):
            for u in range(2):
                for s in range(2):
                    sv = sidx[s][pl.ds(j * 32 + u * 16, 16)]
                    vals = plsc.load_gather(qv[s], [sv])
                    dv = didx[s][pl.ds(j * 32 + u * 16, 16)]
                    plsc.addupdate_scatter(accs[s], [dv], vals)
            return c
        lax.fori_loop(0, tc // 32, eb, 0)
        for s in range(2):
            pltpu.sync_copy(accs[s], out_hbm.at[wid, s])

    return edge_kernel


def _mv_kernel(dx_ref, px_ref, wdp_ref, w1_ref, wpd_ref, w2_ref, s_ref):
    u1 = jnp.dot(wdp_ref[...], w1_ref[...], preferred_element_type=jnp.float32)
    u2 = jnp.dot(wpd_ref[...], w2_ref[...], preferred_element_type=jnp.float32)
    s_ref[0, :] = jnp.dot(dx_ref[...], u1, preferred_element_type=jnp.float32)[:, 0]
    s_ref[1, :] = jnp.dot(px_ref[...], u2, preferred_element_type=jnp.float32)[:, 0]


def _q_kernel(s_ref, deg_ref, q_ref):
    deg = jnp.sum(deg_ref[...], axis=0)
    r = lax.rsqrt(jnp.maximum(deg, 1.0))
    q_ref[0, :] = s_ref[0] * r[0]
    q_ref[1, :] = s_ref[1] * r[2]
    q_ref[2, :] = r[1]
    q_ref[3, :] = r[3]


def _fin_kernel(t_ref, q_ref, bdp_ref, bpd_ref, w1_ref, w2_ref, bl_ref, o_ref):
    c1 = (jnp.sum(bdp_ref[...] * w1_ref[...])
          + jnp.sum(bpd_ref[...] * w2_ref[...]) + bl_ref[0, 0])
    t = jnp.sum(t_ref[...], axis=0)
    z = t[0] * q_ref[2, :] + t[1] * q_ref[3, :] + c1
    o_ref[...] = 1.0 / (1.0 + jnp.exp(-z))


def kernel(drug_x, protein_x, edge_index, rev_edge_index, W_drug_lin,
           b_drug_lin, W_protein_lin, b_protein_lin, conv_W_dp, conv_b_dp,
           conv_W_pd, conv_b_pd, W_link, b_link):
    n = drug_x.shape[0]
    d_h = conv_W_dp.shape[2]
    e = edge_index.shape[1]
    tc = -(-e // (NT * LCH)) * LCH
    epad = NT * tc

    w1 = W_link[:d_h]
    w2 = W_link[d_h:]
    wdp = conv_W_dp[-1]
    wpd = conv_W_pd[-1]
    def prep(v):
        pad = jnp.full((epad - e,), n, jnp.int32)
        return jnp.concatenate([v.astype(jnp.int32), pad]).reshape(NT, tc)

    idx_all = jnp.stack([prep(edge_index[0]), prep(edge_index[1]),
                         prep(rev_edge_index[0]), prep(rev_edge_index[1])])

    deg_part = _make_deg_kernel(tc)(idx_all)

    blk = 1024
    nb = NACC // blk
    s = pl.pallas_call(
        _mv_kernel,
        grid=(nb,),
        in_specs=[
            pl.BlockSpec((blk, drug_x.shape[1]), lambda i: (i, 0)),
            pl.BlockSpec((blk, protein_x.shape[1]), lambda i: (i, 0)),
            pl.BlockSpec(wdp.shape, lambda i: (0, 0)),
            pl.BlockSpec(w1.shape, lambda i: (0, 0)),
            pl.BlockSpec(wpd.shape, lambda i: (0, 0)),
            pl.BlockSpec(w2.shape, lambda i: (0, 0)),
        ],
        out_specs=pl.BlockSpec((2, blk), lambda i: (0, i)),
        out_shape=jax.ShapeDtypeStruct((2, NACC), jnp.float32),
    )(drug_x, protein_x, wdp, w1, wpd, w2)

    q = pl.pallas_call(
        _q_kernel,
        grid=(nb,),
        in_specs=[
            pl.BlockSpec((2, blk), lambda i: (0, i)),
            pl.BlockSpec((NT, 4, blk), lambda i: (0, 0, i)),
        ],
        out_specs=pl.BlockSpec((4, blk), lambda i: (0, i)),
        out_shape=jax.ShapeDtypeStruct((4, NACC), jnp.float32),
    )(s, deg_part)

    t_part = _make_edge_kernel(tc)(q[:2], idx_all)

    out_full = pl.pallas_call(
        _fin_kernel,
        out_shape=jax.ShapeDtypeStruct((NACC,), jnp.float32),
    )(t_part, q,
      conv_b_dp[-1].reshape(2, d_h // 2), conv_b_pd[-1].reshape(2, d_h // 2),
      w1.reshape(2, d_h // 2), w2.reshape(2, d_h // 2),
      b_link.reshape(1, 1))

    return out_full[:n].reshape(n, 1)
